# Initial kernel scaffold; baseline (speedup 1.0000x reference)
import jax
import jax.numpy as jnp
from jax.experimental import pallas as pl


def kernel(x, Wg, W1, b1, W2, b2):
    raise NotImplementedError("write your pallas kernel here")


# speedup vs baseline: 1.1920x; 1.1920x over previous
import functools

import jax
import jax.numpy as jnp
from jax.experimental import pallas as pl
from jax.experimental.pallas import tpu as pltpu

B, S, D = 1, 2048, 768
E, K, H = 8, 2, 3072
T = B * S
BT = 256
NT = T // BT


def _gating_kernel(tok_ref, wg_ref, gate_ref):
    logits = jnp.dot(tok_ref[...], wg_ref[...],
                     preferred_element_type=jnp.float32)
    m = jnp.max(logits, axis=-1, keepdims=True)
    ex = jnp.exp(logits - m)
    probs = ex / jnp.sum(ex, axis=-1, keepdims=True)

    eidx = jax.lax.broadcasted_iota(jnp.int32, (T, E), 1)
    big = jnp.int32(E + 1)

    v1 = jnp.max(probs, axis=-1, keepdims=True)
    i1 = jnp.min(jnp.where(probs == v1, eidx, big), axis=-1, keepdims=True)
    probs2 = jnp.where(eidx == i1, -jnp.inf, probs)
    v2 = jnp.max(probs2, axis=-1, keepdims=True)
    i2 = jnp.min(jnp.where(probs2 == v2, eidx, big), axis=-1, keepdims=True)

    s = v1 + v2
    gate_ref[...] = (jnp.where(eidx == i1, v1 / s, 0.0)
                     + jnp.where(eidx == i2, v2 / s, 0.0))


def _ffn_kernel(tok_ref, w1_ref, b1_ref, w2_ref, b2_ref, gate_ref, y_ref):
    e = pl.program_id(0)
    t = pl.program_id(1)

    xb = tok_ref[pl.ds(t * BT, BT), :]
    h = jnp.dot(xb, w1_ref[0], preferred_element_type=jnp.float32)
    h = h + b1_ref[0]
    a = jax.nn.gelu(h)
    o = jnp.dot(a, w2_ref[0], preferred_element_type=jnp.float32)
    o = o + b2_ref[0]

    gb = gate_ref[pl.ds(t * BT, BT), :]
    eidx = jax.lax.broadcasted_iota(jnp.int32, (BT, E), 1)
    g = jnp.sum(jnp.where(eidx == e, gb, 0.0), axis=1, keepdims=True)
    contrib = o * g

    @pl.when(e == 0)
    def _():
        y_ref[pl.ds(t * BT, BT), :] = contrib

    @pl.when(e > 0)
    def _():
        y_ref[pl.ds(t * BT, BT), :] = y_ref[pl.ds(t * BT, BT), :] + contrib


def kernel(x, Wg, W1, b1, W2, b2):
    tok = x.reshape(T, D)

    gate = pl.pallas_call(
        _gating_kernel,
        out_shape=jax.ShapeDtypeStruct((T, E), jnp.float32),
    )(tok, Wg)

    y = pl.pallas_call(
        _ffn_kernel,
        grid=(E, NT),
        in_specs=[
            pl.BlockSpec((T, D), lambda e, t: (0, 0)),
            pl.BlockSpec((1, D, H), lambda e, t: (e, 0, 0)),
            pl.BlockSpec((1, 1, H), lambda e, t: (e, 0, 0)),
            pl.BlockSpec((1, H, D), lambda e, t: (e, 0, 0)),
            pl.BlockSpec((1, 1, D), lambda e, t: (e, 0, 0)),
            pl.BlockSpec((T, E), lambda e, t: (0, 0)),
        ],
        out_specs=pl.BlockSpec((T, D), lambda e, t: (0, 0)),
        out_shape=jax.ShapeDtypeStruct((T, D), jnp.float32),
    )(tok, W1, b1.reshape(E, 1, H), W2, b2.reshape(E, 1, D), gate)

    return y.reshape(B, S, D)
